# Initial kernel scaffold; baseline (speedup 1.0000x reference)
import functools

import jax
import jax.numpy as jnp
from jax import lax
from jax.experimental import pallas as pl
from jax.experimental.pallas import tpu as pltpu
from jax.experimental.pallas import tpu_sc as plsc

N = 10000
D = 128
E = 320000
NC = 2
NS = 16
NW = NC * NS
CHUNK = 125
CPW = E // (NW * CHUNK)
RPT = N // NS

_MESH = plsc.VectorSubcoreMesh(core_axis_name="c", subcore_axis_name="s")


@functools.partial(
    pl.kernel,
    out_type=jax.ShapeDtypeStruct((NC, 2, N), jnp.float32),
    mesh=_MESH,
    scratch_types=[
        pltpu.VMEM((CPW, CHUNK), jnp.int32),
        pltpu.VMEM((CPW, CHUNK), jnp.int32),
        pltpu.VMEM((128,), jnp.float32),
        pltpu.VMEM((1024,), jnp.float32),
        pltpu.VMEM_SHARED((N,), jnp.float32),
        pltpu.VMEM_SHARED((N,), jnp.float32),
        pltpu.SemaphoreType.DMA,
    ],
)
def _deg_kernel(src_hbm, dst_hbm, out_hbm, src_v, dst_v, ones_v, buf_v,
                hist_s, hist_d, sem):
    cid = lax.axis_index("c")
    sid = lax.axis_index("s")
    wid = cid * NS + sid

    for i in range(8):
        ones_v[pl.ds(i * 16, 16)] = jnp.ones((16,), jnp.float32)
    for i in range(64):
        buf_v[pl.ds(i * 16, 16)] = jnp.zeros((16,), jnp.float32)

    for k in range(20):
        hist = hist_s if k < 10 else hist_d
        off = (k % 10) * 1000

        @pl.when(sid == (k % 16))
        def _(hist=hist, off=off):
            pltpu.sync_copy(buf_v.at[pl.ds(0, 1000)], hist.at[pl.ds(off, 1000)])

    plsc.subcore_barrier()

    pltpu.sync_copy(src_hbm.at[wid], src_v)
    pltpu.sync_copy(dst_hbm.at[wid], dst_v)

    def body(j, carry):
        pltpu.sync_copy(ones_v.at[pl.ds(0, CHUNK)], hist_s.at[src_v.at[j]],
                        add=True)
        pltpu.sync_copy(ones_v.at[pl.ds(0, CHUNK)], hist_d.at[dst_v.at[j]],
                        add=True)
        return carry

    lax.fori_loop(0, CPW, body, 0)

    plsc.subcore_barrier()

    for k in range(20):
        hist = hist_s if k < 10 else hist_d
        which = 0 if k < 10 else 1
        off = (k % 10) * 1000

        @pl.when(sid == (k % 16))
        def _(hist=hist, which=which, off=off):
            pltpu.sync_copy(hist.at[pl.ds(off, 1000)], buf_v.at[pl.ds(0, 1000)])
            pltpu.sync_copy(buf_v.at[pl.ds(0, 1000)],
                            out_hbm.at[cid, which, pl.ds(off, 1000)])


@functools.partial(
    pl.kernel,
    out_type=jax.ShapeDtypeStruct((NC, N, D), jnp.float32),
    mesh=_MESH,
    scratch_types=[
        pltpu.VMEM((CPW, CHUNK), jnp.int32),
        pltpu.VMEM((CPW, CHUNK), jnp.int32),
        pltpu.VMEM((CHUNK, D), jnp.float32),
        pltpu.VMEM((CHUNK, D), jnp.float32),
        pltpu.VMEM_SHARED((N, D), jnp.float32),
        pltpu.SemaphoreType.DMA,
    ],
)
def _spmm_kernel(src_hbm, dst_hbm, feat_hbm, out_hbm, src_v, dst_v, rows_v,
                 zrow_v, accum, sem):
    cid = lax.axis_index("c")
    sid = lax.axis_index("s")
    wid = cid * NS + sid

    def zfill(j, carry):
        for c in range(8):
            zrow_v[j, pl.ds(c * 16, 16)] = jnp.zeros((16,), jnp.float32)
        return carry

    lax.fori_loop(0, CHUNK, zfill, 0)

    def zacc(k, carry):
        pltpu.sync_copy(zrow_v, accum.at[pl.ds(sid * RPT + k * CHUNK, CHUNK)])
        return carry

    lax.fori_loop(0, 5, zacc, 0)
    plsc.subcore_barrier()

    pltpu.sync_copy(src_hbm.at[wid], src_v)
    pltpu.sync_copy(dst_hbm.at[wid], dst_v)

    def body(j, carry):
        pltpu.async_copy(feat_hbm.at[src_v.at[j]], rows_v, sem).wait()
        pltpu.sync_copy(rows_v, accum.at[dst_v.at[j]], add=True)
        return carry

    lax.fori_loop(0, CPW, body, 0)

    plsc.subcore_barrier()

    def copyout(k, carry):
        off = sid * RPT + k * CHUNK
        pltpu.sync_copy(accum.at[pl.ds(off, CHUNK)], rows_v)
        pltpu.sync_copy(rows_v, out_hbm.at[cid, pl.ds(off, CHUNK)])
        return carry

    lax.fori_loop(0, 5, copyout, 0)


RB = 1000


def _scale_body(feat_ref, degt_ref, out_ref):
    od = jnp.sum(degt_ref[:, :, 0], axis=1, keepdims=True)
    norm = lax.rsqrt(jnp.maximum(od, 1.0))
    out_ref[...] = feat_ref[...] * norm


_scale = pl.pallas_call(
    _scale_body,
    grid=(N // RB,),
    in_specs=[
        pl.BlockSpec((RB, D), lambda i: (i, 0)),
        pl.BlockSpec((RB, NC, 2), lambda i: (i, 0, 0)),
    ],
    out_specs=pl.BlockSpec((RB, D), lambda i: (i, 0)),
    out_shape=jax.ShapeDtypeStruct((N, D), jnp.float32),
)


def _out_body(accp_ref, w_ref, degt_ref, out_ref):
    acc = accp_ref[0] + accp_ref[1]
    y = jnp.dot(acc, w_ref[...], preferred_element_type=jnp.float32)
    ind = jnp.sum(degt_ref[:, :, 1], axis=1, keepdims=True)
    out_ref[...] = y * lax.rsqrt(jnp.maximum(ind, 1.0))


_combine = pl.pallas_call(
    _out_body,
    grid=(N // RB,),
    in_specs=[
        pl.BlockSpec((NC, RB, D), lambda i: (0, i, 0)),
        pl.BlockSpec((D, D), lambda i: (0, 0)),
        pl.BlockSpec((RB, NC, 2), lambda i: (i, 0, 0)),
    ],
    out_specs=pl.BlockSpec((RB, D), lambda i: (i, 0)),
    out_shape=jax.ShapeDtypeStruct((N, D), jnp.float32),
)


@jax.jit
def kernel(feat, edge_index, weight):
    src = edge_index[0].astype(jnp.int32).reshape(NW, CPW, CHUNK)
    dst = edge_index[1].astype(jnp.int32).reshape(NW, CPW, CHUNK)
    degp = _deg_kernel(src, dst)
    degt = jnp.transpose(degp, (2, 0, 1))
    feat_src = _scale(feat, degt)
    accp = _spmm_kernel(src, dst, feat_src)
    return _combine(accp, weight, degt)


# speedup vs baseline: 6.3888x; 6.3888x over previous
import functools

import jax
import jax.numpy as jnp
from jax import lax
from jax.experimental import pallas as pl
from jax.experimental.pallas import tpu as pltpu
from jax.experimental.pallas import tpu_sc as plsc

N = 10000
D = 128
E = 320000
NC = 2
NS = 16
NW = NC * NS
CHUNK = 125
CPW = E // (NW * CHUNK)
COCH = 80
NCO = N // COCH
GB = 16
NG = CPW // GB

_MESH = plsc.VectorSubcoreMesh(core_axis_name="c", subcore_axis_name="s")


@functools.partial(
    pl.kernel,
    out_type=[
        jax.ShapeDtypeStruct((NC * N,), jnp.float32),
        jax.ShapeDtypeStruct((NC * N,), jnp.float32),
    ],
    mesh=_MESH,
    scratch_types=[
        pltpu.VMEM((CPW, CHUNK), jnp.int32),
        pltpu.VMEM((CPW, CHUNK), jnp.int32),
        pltpu.VMEM((128,), jnp.float32),
        pltpu.VMEM((1024,), jnp.float32),
        pltpu.VMEM_SHARED((N,), jnp.float32),
        pltpu.VMEM_SHARED((N,), jnp.float32),
        pltpu.SemaphoreType.DMA,
    ],
)
def _deg_kernel(src_hbm, dst_hbm, out_s_hbm, out_d_hbm, src_v, dst_v, ones_v,
                buf_v, hist_s, hist_d, sem):
    cid = lax.axis_index("c")
    sid = lax.axis_index("s")
    wid = cid * NS + sid

    for i in range(8):
        ones_v[pl.ds(i * 16, 16)] = jnp.ones((16,), jnp.float32)
    for i in range(64):
        buf_v[pl.ds(i * 16, 16)] = jnp.zeros((16,), jnp.float32)

    for k in range(20):
        hist = hist_s if k < 10 else hist_d
        off = (k % 10) * 1000

        @pl.when(sid == (k % 16))
        def _(hist=hist, off=off):
            pltpu.sync_copy(buf_v.at[pl.ds(0, 1000)], hist.at[pl.ds(off, 1000)])

    plsc.subcore_barrier()

    pltpu.sync_copy(src_hbm.at[wid], src_v)
    pltpu.sync_copy(dst_hbm.at[wid], dst_v)

    def body(j, carry):
        pltpu.sync_copy(ones_v.at[pl.ds(0, CHUNK)], hist_s.at[src_v.at[j]],
                        add=True)
        pltpu.sync_copy(ones_v.at[pl.ds(0, CHUNK)], hist_d.at[dst_v.at[j]],
                        add=True)
        return carry

    lax.fori_loop(0, CPW, body, 0)

    plsc.subcore_barrier()

    for k in range(20):
        hist = hist_s if k < 10 else hist_d
        out = out_s_hbm if k < 10 else out_d_hbm
        off = (k % 10) * 1000

        @pl.when(sid == (k % 16))
        def _(hist=hist, out=out, off=off):
            pltpu.sync_copy(hist.at[pl.ds(off, 1000)], buf_v.at[pl.ds(0, 1000)])
            pltpu.sync_copy(buf_v.at[pl.ds(0, 1000)],
                            out.at[pl.ds(cid * N + off, 1000)])


@functools.partial(
    pl.kernel,
    out_type=jax.ShapeDtypeStruct((NC, N, D), jnp.float32),
    mesh=_MESH,
    scratch_types=[
        pltpu.VMEM((GB, CHUNK), jnp.int32),
        pltpu.VMEM((GB, CHUNK), jnp.int32),
        pltpu.VMEM((CHUNK, D), jnp.float32),
        pltpu.VMEM((COCH, D), jnp.float32),
        pltpu.VMEM_SHARED((N, D), jnp.float32),
        pltpu.SemaphoreType.DMA,
    ],
)
def _spmm_kernel(src_hbm, dst_hbm, feat_hbm, out_hbm, src_v, dst_v, rows_v,
                 zrow_v, accum, sem):
    cid = lax.axis_index("c")
    sid = lax.axis_index("s")
    wid = cid * NS + sid

    def zfill(j, carry):
        for c in range(8):
            zrow_v[j, pl.ds(c * 16, 16)] = jnp.zeros((16,), jnp.float32)
        return carry

    lax.fori_loop(0, COCH, zfill, 0)

    def zacc(k, carry):
        pltpu.sync_copy(zrow_v, accum.at[pl.ds((sid + k * NS) * COCH, COCH)])
        return carry

    lax.fori_loop(0, NCO // NS, zacc, 0)

    @pl.when(sid < NCO % NS)
    def _():
        pltpu.sync_copy(
            zrow_v, accum.at[pl.ds((sid + (NCO // NS) * NS) * COCH, COCH)])

    plsc.subcore_barrier()

    def group(g, carry):
        pltpu.sync_copy(src_hbm.at[wid, pl.ds(g * GB, GB)], src_v)
        pltpu.sync_copy(dst_hbm.at[wid, pl.ds(g * GB, GB)], dst_v)

        def body(j, carry2):
            pltpu.async_copy(feat_hbm.at[src_v.at[j]], rows_v, sem).wait()
            pltpu.sync_copy(rows_v, accum.at[dst_v.at[j]], add=True)
            return carry2

        lax.fori_loop(0, GB, body, 0)
        return carry

    lax.fori_loop(0, NG, group, 0)

    plsc.subcore_barrier()

    def copyout(k, carry):
        off = (sid + k * NS) * COCH
        pltpu.sync_copy(accum.at[pl.ds(off, COCH)], zrow_v)
        pltpu.sync_copy(zrow_v, out_hbm.at[cid, pl.ds(off, COCH)])
        return carry

    lax.fori_loop(0, NCO // NS, copyout, 0)

    @pl.when(sid < NCO % NS)
    def _():
        off = (sid + (NCO // NS) * NS) * COCH
        pltpu.sync_copy(accum.at[pl.ds(off, COCH)], zrow_v)
        pltpu.sync_copy(zrow_v, out_hbm.at[cid, pl.ds(off, COCH)])


RB = 1000


def _scale_body(feat_ref, degt_ref, out_ref):
    od = jnp.sum(degt_ref[:, :, 0], axis=1, keepdims=True)
    norm = lax.rsqrt(jnp.maximum(od, 1.0))
    out_ref[...] = feat_ref[...] * norm


_scale = pl.pallas_call(
    _scale_body,
    grid=(N // RB,),
    in_specs=[
        pl.BlockSpec((RB, D), lambda i: (i, 0)),
        pl.BlockSpec((RB, NC, 2), lambda i: (i, 0, 0)),
    ],
    out_specs=pl.BlockSpec((RB, D), lambda i: (i, 0)),
    out_shape=jax.ShapeDtypeStruct((N, D), jnp.float32),
)


def _out_body(accp_ref, w_ref, degt_ref, out_ref):
    acc = accp_ref[0] + accp_ref[1]
    y = jnp.dot(acc, w_ref[...], preferred_element_type=jnp.float32)
    ind = jnp.sum(degt_ref[:, :, 1], axis=1, keepdims=True)
    out_ref[...] = y * lax.rsqrt(jnp.maximum(ind, 1.0))


_combine = pl.pallas_call(
    _out_body,
    grid=(N // RB,),
    in_specs=[
        pl.BlockSpec((NC, RB, D), lambda i: (0, i, 0)),
        pl.BlockSpec((D, D), lambda i: (0, 0)),
        pl.BlockSpec((RB, NC, 2), lambda i: (i, 0, 0)),
    ],
    out_specs=pl.BlockSpec((RB, D), lambda i: (i, 0)),
    out_shape=jax.ShapeDtypeStruct((N, D), jnp.float32),
)


@jax.jit
def kernel(feat, edge_index, weight):
    src = edge_index[0].astype(jnp.int32).reshape(NW, CPW, CHUNK)
    dst = edge_index[1].astype(jnp.int32).reshape(NW, CPW, CHUNK)
    deg_s, deg_d = _deg_kernel(src, dst)
    degt = jnp.stack(
        [deg_s.reshape(NC, N).T, deg_d.reshape(NC, N).T], axis=-1
    )
    feat_src = _scale(feat, degt)
    accp = _spmm_kernel(src, dst, feat_src)
    return _combine(accp, weight, degt)


# speedup vs baseline: 8.0249x; 1.2561x over previous
import functools

import jax
import jax.numpy as jnp
from jax import lax
from jax.experimental import pallas as pl
from jax.experimental.pallas import tpu as pltpu
from jax.experimental.pallas import tpu_sc as plsc

N = 10000
D = 128
E = 320000
NC = 2
NS = 16
NW = NC * NS
CHUNK = 125
CPW = E // (NW * CHUNK)
COCH = 80
NCO = N // COCH
GB = 16
NG = CPW // GB

_MESH = plsc.VectorSubcoreMesh(core_axis_name="c", subcore_axis_name="s")


@functools.partial(
    pl.kernel,
    out_type=[
        jax.ShapeDtypeStruct((NC * N,), jnp.float32),
        jax.ShapeDtypeStruct((NC * N,), jnp.float32),
    ],
    mesh=_MESH,
    scratch_types=[
        pltpu.VMEM((CPW, CHUNK), jnp.int32),
        pltpu.VMEM((CPW, CHUNK), jnp.int32),
        pltpu.VMEM((128,), jnp.float32),
        pltpu.VMEM((1024,), jnp.float32),
        pltpu.VMEM_SHARED((N,), jnp.float32),
        pltpu.VMEM_SHARED((N,), jnp.float32),
        pltpu.SemaphoreType.DMA,
    ],
)
def _deg_kernel(src_hbm, dst_hbm, out_s_hbm, out_d_hbm, src_v, dst_v, ones_v,
                buf_v, hist_s, hist_d, sem):
    cid = lax.axis_index("c")
    sid = lax.axis_index("s")
    wid = cid * NS + sid

    for i in range(8):
        ones_v[pl.ds(i * 16, 16)] = jnp.ones((16,), jnp.float32)
    for i in range(64):
        buf_v[pl.ds(i * 16, 16)] = jnp.zeros((16,), jnp.float32)

    for k in range(20):
        hist = hist_s if k < 10 else hist_d
        off = (k % 10) * 1000

        @pl.when(sid == (k % 16))
        def _(hist=hist, off=off):
            pltpu.sync_copy(buf_v.at[pl.ds(0, 1000)], hist.at[pl.ds(off, 1000)])

    plsc.subcore_barrier()

    pltpu.sync_copy(src_hbm.at[wid], src_v)
    pltpu.sync_copy(dst_hbm.at[wid], dst_v)

    def body(j, carry):
        pltpu.sync_copy(ones_v.at[pl.ds(0, CHUNK)], hist_s.at[src_v.at[j]],
                        add=True)
        pltpu.sync_copy(ones_v.at[pl.ds(0, CHUNK)], hist_d.at[dst_v.at[j]],
                        add=True)
        return carry

    lax.fori_loop(0, CPW, body, 0)

    plsc.subcore_barrier()

    for k in range(20):
        hist = hist_s if k < 10 else hist_d
        out = out_s_hbm if k < 10 else out_d_hbm
        off = (k % 10) * 1000

        @pl.when(sid == (k % 16))
        def _(hist=hist, out=out, off=off):
            pltpu.sync_copy(hist.at[pl.ds(off, 1000)], buf_v.at[pl.ds(0, 1000)])
            pltpu.sync_copy(buf_v.at[pl.ds(0, 1000)],
                            out.at[pl.ds(cid * N + off, 1000)])


@functools.partial(
    pl.kernel,
    out_type=jax.ShapeDtypeStruct((NC, N, D), jnp.float32),
    mesh=_MESH,
    scratch_types=[
        pltpu.VMEM((GB, CHUNK), jnp.int32),
        pltpu.VMEM((GB, CHUNK), jnp.int32),
        pltpu.VMEM((CHUNK, D), jnp.float32),
        pltpu.VMEM((CHUNK, D), jnp.float32),
        pltpu.VMEM((COCH, D), jnp.float32),
        pltpu.VMEM_SHARED((N, D), jnp.float32),
        pltpu.SemaphoreType.DMA,
        pltpu.SemaphoreType.DMA,
    ],
)
def _spmm_kernel(src_hbm, dst_hbm, feat_hbm, out_hbm, src_v, dst_v, rows_a,
                 rows_b, zrow_v, accum, sem_a, sem_b):
    cid = lax.axis_index("c")
    sid = lax.axis_index("s")
    wid = cid * NS + sid

    def zfill(j, carry):
        for c in range(8):
            zrow_v[j, pl.ds(c * 16, 16)] = jnp.zeros((16,), jnp.float32)
        return carry

    lax.fori_loop(0, COCH, zfill, 0)

    def zacc(k, carry):
        pltpu.sync_copy(zrow_v, accum.at[pl.ds((sid + k * NS) * COCH, COCH)])
        return carry

    lax.fori_loop(0, NCO // NS, zacc, 0)

    @pl.when(sid < NCO % NS)
    def _():
        pltpu.sync_copy(
            zrow_v, accum.at[pl.ds((sid + (NCO // NS) * NS) * COCH, COCH)])

    plsc.subcore_barrier()

    def group(g, carry):
        pltpu.sync_copy(src_hbm.at[wid, pl.ds(g * GB, GB)], src_v)
        pltpu.sync_copy(dst_hbm.at[wid, pl.ds(g * GB, GB)], dst_v)

        pltpu.async_copy(feat_hbm.at[src_v.at[0]], rows_a, sem_a)

        def pair(p, carry2):
            j0 = 2 * p
            pltpu.async_copy(feat_hbm.at[src_v.at[j0 + 1]], rows_b, sem_b)
            pltpu.make_async_copy(feat_hbm.at[src_v.at[j0]], rows_a,
                                  sem_a).wait()
            pltpu.sync_copy(rows_a, accum.at[dst_v.at[j0]], add=True)

            @pl.when(j0 + 2 < GB)
            def _():
                pltpu.async_copy(feat_hbm.at[src_v.at[j0 + 2]], rows_a, sem_a)

            pltpu.make_async_copy(feat_hbm.at[src_v.at[j0 + 1]], rows_b,
                                  sem_b).wait()
            pltpu.sync_copy(rows_b, accum.at[dst_v.at[j0 + 1]], add=True)
            return carry2

        lax.fori_loop(0, GB // 2, pair, 0)
        return carry

    lax.fori_loop(0, NG, group, 0)

    plsc.subcore_barrier()

    def copyout(k, carry):
        off = (sid + k * NS) * COCH
        pltpu.sync_copy(accum.at[pl.ds(off, COCH)], zrow_v)
        pltpu.sync_copy(zrow_v, out_hbm.at[cid, pl.ds(off, COCH)])
        return carry

    lax.fori_loop(0, NCO // NS, copyout, 0)

    @pl.when(sid < NCO % NS)
    def _():
        off = (sid + (NCO // NS) * NS) * COCH
        pltpu.sync_copy(accum.at[pl.ds(off, COCH)], zrow_v)
        pltpu.sync_copy(zrow_v, out_hbm.at[cid, pl.ds(off, COCH)])


RB = 1000


def _scale_body(feat_ref, degt_ref, out_ref):
    od = jnp.sum(degt_ref[:, :, 0], axis=1, keepdims=True)
    norm = lax.rsqrt(jnp.maximum(od, 1.0))
    out_ref[...] = feat_ref[...] * norm


_scale = pl.pallas_call(
    _scale_body,
    grid=(N // RB,),
    in_specs=[
        pl.BlockSpec((RB, D), lambda i: (i, 0)),
        pl.BlockSpec((RB, NC, 2), lambda i: (i, 0, 0)),
    ],
    out_specs=pl.BlockSpec((RB, D), lambda i: (i, 0)),
    out_shape=jax.ShapeDtypeStruct((N, D), jnp.float32),
)


def _out_body(accp_ref, w_ref, degt_ref, out_ref):
    acc = accp_ref[0] + accp_ref[1]
    y = jnp.dot(acc, w_ref[...], preferred_element_type=jnp.float32)
    ind = jnp.sum(degt_ref[:, :, 1], axis=1, keepdims=True)
    out_ref[...] = y * lax.rsqrt(jnp.maximum(ind, 1.0))


_combine = pl.pallas_call(
    _out_body,
    grid=(N // RB,),
    in_specs=[
        pl.BlockSpec((NC, RB, D), lambda i: (0, i, 0)),
        pl.BlockSpec((D, D), lambda i: (0, 0)),
        pl.BlockSpec((RB, NC, 2), lambda i: (i, 0, 0)),
    ],
    out_specs=pl.BlockSpec((RB, D), lambda i: (i, 0)),
    out_shape=jax.ShapeDtypeStruct((N, D), jnp.float32),
)


@jax.jit
def kernel(feat, edge_index, weight):
    src = edge_index[0].astype(jnp.int32).reshape(NW, CPW, CHUNK)
    dst = edge_index[1].astype(jnp.int32).reshape(NW, CPW, CHUNK)
    deg_s, deg_d = _deg_kernel(src, dst)
    degt = jnp.stack(
        [deg_s.reshape(NC, N).T, deg_d.reshape(NC, N).T], axis=-1
    )
    feat_src = _scale(feat, degt)
    accp = _spmm_kernel(src, dst, feat_src)
    return _combine(accp, weight, degt)
